# baseline (speedup 1.0000x reference)
import functools

import jax
import jax.numpy as jnp
import numpy as np
from jax import lax
from jax.experimental import pallas as pl
from jax.experimental.pallas import tpu as pltpu
from jax.experimental.pallas import tpu_sc as plsc

N = 10000
E = 320000
D = 128
NC = 2
NS = 16
LANES = 16
CHUNK = 80
SEG_CHUNKS = 25
SEG_EDGES = SEG_CHUNKS * CHUNK
BN = 1000

_PERM = np.concatenate([
    np.concatenate([32 * q + np.arange(0, 32, 2),
                    32 * q + np.arange(1, 32, 2)])
    for q in range(D // 32)
])


def _splat_lane(vec, i):
    idx = jnp.full((LANES, 1), i, jnp.int32)
    dn = lax.GatherDimensionNumbers(offset_dims=(), collapsed_slice_dims=(0,),
                                    start_index_map=(0,))
    return lax.gather(vec, idx, dn, (1,),
                      mode=lax.GatherScatterMode.PROMISE_IN_BOUNDS)


def _sc_aggregate_body(dual, n_chunks, x_hbm, esrc_hbm, edst_hbm,
                       ew_hbm, out_hbm, src_all, dst_all, w_all,
                       idx_a, idx_b, dst_a, dst_b, r16_a, r16_b,
                       out_a, out_b, acc, gsem_a, gsem_b, ssem_a, ssem_b,
                       stage_sem):
    c = lax.axis_index("c")
    s = lax.axis_index("s")

    zero = jnp.zeros((LANES,), jnp.float32)
    for r in range(CHUNK):
        for j in range(D // LANES):
            out_a[r, pl.ds(j * LANES, LANES)] = zero
    rows_pt = 624
    rem_rows = N - NS * rows_pt
    r0 = pl.multiple_of(s * rows_pt, 8)
    nfull = rows_pt // CHUNK
    tail = rows_pt - nfull * CHUNK
    for k in range(nfull):
        pltpu.sync_copy(out_a, acc.at[pl.ds(r0 + k * CHUNK, CHUNK)])
    if tail:
        pltpu.sync_copy(out_a.at[pl.ds(0, tail)],
                        acc.at[pl.ds(r0 + nfull * CHUNK, tail)])

    @pl.when(s == NS - 1)
    def _():
        pltpu.sync_copy(out_a.at[pl.ds(0, rem_rows)],
                        acc.at[pl.ds(NS * rows_pt, rem_rows)])

    plsc.subcore_barrier()

    ept = n_chunks * CHUNK
    if dual:
        tile_base = s * ept
        off = jnp.broadcast_to((c * N).astype(jnp.int32), (LANES,))
    else:
        tile_base = (c * NS + s) * ept

    n_segs = n_chunks // SEG_CHUNKS
    n_pairs = SEG_CHUNKS // 2

    def _prep_idx(ci, idx_v):
        base = pl.multiple_of(ci * CHUNK, 8)
        for g in range(CHUNK // LANES):
            sl = pl.ds(g * LANES, LANES)
            slb = pl.ds(base + g * LANES, LANES)
            if dual:
                idx_v[sl] = src_all[slb] + off
            else:
                idx_v[sl] = src_all[slb]

    def _prep_dst(ci, dst_v):
        base = pl.multiple_of(ci * CHUNK, 8)
        for g in range(CHUNK // LANES):
            sl = pl.ds(g * LANES, LANES)
            slb = pl.ds(base + g * LANES, LANES)
            dst_v[sl] = dst_all[slb]

    def _scale(ci, r16_v, out_v):
        base = pl.multiple_of(ci * CHUNK, 8)
        for g in range(CHUNK // LANES):
            w16 = w_all[pl.ds(base + g * LANES, LANES)]
            for i in range(LANES):
                w = _splat_lane(w16, i)
                r = g * LANES + i
                for j in range(D // LANES):
                    sl = pl.ds(j * LANES, LANES)
                    out_v[r, sl] = r16_v[r, sl] * w

    def _drain(out_v, dst_v, ssem):
        pltpu.make_async_copy(out_v, acc.at[dst_v], ssem).wait()

    def _stage(si, sem_or_none):
        sb = pl.multiple_of(tile_base + si * SEG_EDGES, 8)
        if sem_or_none is None:
            pltpu.sync_copy(esrc_hbm.at[pl.ds(sb, SEG_EDGES)], src_all)
            pltpu.sync_copy(edst_hbm.at[pl.ds(sb, SEG_EDGES)], dst_all)
            pltpu.sync_copy(ew_hbm.at[pl.ds(sb, SEG_EDGES)], w_all)
        else:
            pltpu.async_copy(esrc_hbm.at[pl.ds(sb, SEG_EDGES)], src_all,
                             sem_or_none)
            pltpu.async_copy(edst_hbm.at[pl.ds(sb, SEG_EDGES)], dst_all,
                             sem_or_none)
            pltpu.async_copy(ew_hbm.at[pl.ds(sb, SEG_EDGES)], w_all,
                             sem_or_none)

    def _stage_wait(si, sem):
        sb = pl.multiple_of(tile_base + si * SEG_EDGES, 8)
        pltpu.make_async_copy(esrc_hbm.at[pl.ds(sb, SEG_EDGES)], src_all,
                              sem).wait()
        pltpu.make_async_copy(edst_hbm.at[pl.ds(sb, SEG_EDGES)], dst_all,
                              sem).wait()
        pltpu.make_async_copy(ew_hbm.at[pl.ds(sb, SEG_EDGES)], w_all,
                              sem).wait()

    _stage(0, None)

    def seg_body(si, carry):
        @pl.when(si > 0)
        def _():
            _stage_wait(si, stage_sem)

        _prep_idx(0, idx_a)
        pltpu.async_copy(x_hbm.at[idx_a], r16_a, gsem_a)
        _prep_idx(1, idx_b)
        pltpu.async_copy(x_hbm.at[idx_b], r16_b, gsem_b)

        def pair_body(pi, carry2):
            @pl.when((pi > 0) | (si > 0))
            def _():
                _drain(out_a, dst_a, ssem_a)

            pltpu.make_async_copy(x_hbm.at[idx_a], r16_a, gsem_a).wait()
            _scale(2 * pi, r16_a, out_a)
            _prep_dst(2 * pi, dst_a)
            pltpu.async_copy(out_a, acc.at[dst_a], ssem_a, add=True)

            @pl.when(pi + 1 < n_pairs)
            def _():
                _prep_idx(2 * (pi + 1), idx_a)
                pltpu.async_copy(x_hbm.at[idx_a], r16_a, gsem_a)

            @pl.when(pi > 0)
            def _():
                _drain(out_b, dst_b, ssem_b)

            pltpu.make_async_copy(x_hbm.at[idx_b], r16_b, gsem_b).wait()
            _scale(2 * pi + 1, r16_b, out_b)
            _prep_dst(2 * pi + 1, dst_b)
            pltpu.async_copy(out_b, acc.at[dst_b], ssem_b, add=True)

            @pl.when(pi + 1 < n_pairs)
            def _():
                _prep_idx(2 * (pi + 1) + 1, idx_b)
                pltpu.async_copy(x_hbm.at[idx_b], r16_b, gsem_b)

            return carry2

        lax.fori_loop(0, n_pairs, pair_body, 0)
        _drain(out_a, dst_a, ssem_a)
        _drain(out_b, dst_b, ssem_b)
        rem_ci = SEG_CHUNKS - 1
        _prep_idx(rem_ci, idx_a)
        pltpu.async_copy(x_hbm.at[idx_a], r16_a, gsem_a).wait()
        _scale(rem_ci, r16_a, out_a)
        _prep_dst(rem_ci, dst_a)
        pltpu.async_copy(out_a, acc.at[dst_a], ssem_a, add=True)

        @pl.when(si + 1 < n_segs)
        def _():
            _stage(si + 1, stage_sem)

        return carry

    lax.fori_loop(0, n_segs, seg_body, 0)
    _drain(out_a, dst_a, ssem_a)
    plsc.subcore_barrier()
    pltpu.sync_copy(acc.at[pl.ds(r0, rows_pt)],
                    out_hbm.at[c, pl.ds(r0, rows_pt)])

    @pl.when(s == NS - 1)
    def _():
        pltpu.sync_copy(acc.at[pl.ds(NS * rows_pt, rem_rows)],
                        out_hbm.at[c, pl.ds(NS * rows_pt, rem_rows)])


_SC_SCRATCH = [
    pltpu.VMEM((SEG_EDGES,), jnp.int32),
    pltpu.VMEM((SEG_EDGES,), jnp.int32),
    pltpu.VMEM((SEG_EDGES,), jnp.float32),
    pltpu.VMEM((CHUNK,), jnp.int32),
    pltpu.VMEM((CHUNK,), jnp.int32),
    pltpu.VMEM((CHUNK,), jnp.int32),
    pltpu.VMEM((CHUNK,), jnp.int32),
    pltpu.VMEM((CHUNK, D), jnp.float32),
    pltpu.VMEM((CHUNK, D), jnp.float32),
    pltpu.VMEM((CHUNK, D), jnp.float32),
    pltpu.VMEM((CHUNK, D), jnp.float32),
    pltpu.VMEM_SHARED((N, D), jnp.float32),
    pltpu.SemaphoreType.DMA,
    pltpu.SemaphoreType.DMA,
    pltpu.SemaphoreType.DMA,
    pltpu.SemaphoreType.DMA,
    pltpu.SemaphoreType.DMA,
]


def _sc_pass1(XHp, esrc, edst, ew):
    mesh = plsc.VectorSubcoreMesh(core_axis_name="c", subcore_axis_name="s")
    body = functools.partial(_sc_aggregate_body, True, E // NS // CHUNK)
    f = pl.kernel(body,
                  out_type=jax.ShapeDtypeStruct((NC, N, D), jnp.float32),
                  mesh=mesh, scratch_types=_SC_SCRATCH)
    return f(XHp, esrc, edst, ew)


def _sc_pass2(HRp, esrc, edst, ew):
    mesh = plsc.VectorSubcoreMesh(core_axis_name="c", subcore_axis_name="s")
    body = functools.partial(_sc_aggregate_body, False,
                             E // (NC * NS) // CHUNK)
    f = pl.kernel(body,
                  out_type=jax.ShapeDtypeStruct((NC, N, D), jnp.float32),
                  mesh=mesh, scratch_types=_SC_SCRATCH)
    return f(HRp, esrc, edst, ew)


def _tc_gates(aggrXH, X, H, Wzr, bzr):
    def body(axh, x, h, wzr, b, z_out, hr_out):
        a = jnp.concatenate([axh[0], x[...], axh[1], h[...]], axis=1)
        g = jnp.dot(a, wzr[...], preferred_element_type=jnp.float32) + b[...]
        z = jax.nn.sigmoid(g[:, :D])
        r = jax.nn.sigmoid(g[:, D:])
        z_out[...] = z
        hr_out[...] = h[...] * r

    return pl.pallas_call(
        body,
        grid=(N // BN,),
        in_specs=[
            pl.BlockSpec((NC, BN, D), lambda i: (0, i, 0)),
            pl.BlockSpec((BN, D), lambda i: (i, 0)),
            pl.BlockSpec((BN, D), lambda i: (i, 0)),
            pl.BlockSpec((4 * D, 2 * D), lambda i: (0, 0)),
            pl.BlockSpec((1, 2 * D), lambda i: (0, 0)),
        ],
        out_specs=[pl.BlockSpec((BN, D), lambda i: (i, 0))] * 2,
        out_shape=[jax.ShapeDtypeStruct((N, D), jnp.float32)] * 2,
    )(aggrXH, X, H, Wzr, bzr)


def _tc_out(aggrXH, parts, X, HR, H, Z, Wh, bh):
    def body(axh, p, x, hr, h, z, wh, b, out):
        a = jnp.concatenate([axh[0], x[...], p[0], p[1], hr[...]], axis=1)
        g = jnp.dot(a, wh[...], preferred_element_type=jnp.float32) + b[...]
        ht = jnp.tanh(g)
        out[...] = z[...] * h[...] + (1.0 - z[...]) * ht

    return pl.pallas_call(
        body,
        grid=(N // BN,),
        in_specs=[
            pl.BlockSpec((1, BN, D), lambda i: (0, i, 0)),
            pl.BlockSpec((NC, BN, D), lambda i: (0, i, 0)),
            pl.BlockSpec((BN, D), lambda i: (i, 0)),
            pl.BlockSpec((BN, D), lambda i: (i, 0)),
            pl.BlockSpec((BN, D), lambda i: (i, 0)),
            pl.BlockSpec((BN, D), lambda i: (i, 0)),
            pl.BlockSpec((5 * D, D), lambda i: (0, 0)),
            pl.BlockSpec((1, D), lambda i: (0, 0)),
        ],
        out_specs=pl.BlockSpec((BN, D), lambda i: (i, 0)),
        out_shape=jax.ShapeDtypeStruct((N, D), jnp.float32),
    )(aggrXH, parts, X, HR, H, Z, Wh, bh)


def kernel(X, edge_index, edge_weight, H,
           Wl_xz, Wr_xz, b_xz, Wl_hz, Wr_hz, b_hz,
           Wl_xr, Wr_xr, b_xr, Wl_hr, Wr_hr, b_hr,
           Wl_xh, Wr_xh, b_xh, Wl_hh, Wr_hh, b_hh):
    esrc = edge_index[0]
    edst = edge_index[1]
    XHp = jnp.concatenate([X, H], axis=0)
    aggrXH = _sc_pass1(XHp, esrc, edst, edge_weight)

    Wzr = jnp.concatenate([
        jnp.concatenate([Wl_xz, Wl_xr], axis=1),
        jnp.concatenate([Wr_xz, Wr_xr], axis=1),
        jnp.concatenate([Wl_hz, Wl_hr], axis=1),
        jnp.concatenate([Wr_hz, Wr_hr], axis=1),
    ], axis=0)
    bzr = jnp.concatenate([b_xz + b_hz, b_xr + b_hr]).reshape(1, 2 * D)
    Z, HR = _tc_gates(aggrXH, X, H, Wzr, bzr)

    parts = _sc_pass2(HR, esrc, edst, edge_weight)

    Wh = jnp.concatenate([Wl_xh, Wr_xh, Wl_hh, Wl_hh,
                          Wr_hh], axis=0)
    bh = (b_xh + b_hh).reshape(1, D)
    return _tc_out(aggrXH, parts, X, HR, H, Z, Wh, bh)


# speedup vs baseline: 1.0903x; 1.0903x over previous
import functools

import jax
import jax.numpy as jnp
import numpy as np
from jax import lax
from jax.experimental import pallas as pl
from jax.experimental.pallas import tpu as pltpu
from jax.experimental.pallas import tpu_sc as plsc

N = 10000
E = 320000
D = 128
NC = 2
NS = 16
LANES = 16
CHUNK = 80
SEG_CHUNKS = 25
SEG_EDGES = SEG_CHUNKS * CHUNK
BN = 1000

_PERM = np.concatenate([
    np.concatenate([32 * q + np.arange(0, 32, 2),
                    32 * q + np.arange(1, 32, 2)])
    for q in range(D // 32)
])


def _splat_lane(vec, i):
    idx = jnp.full((LANES, 1), i, jnp.int32)
    dn = lax.GatherDimensionNumbers(offset_dims=(), collapsed_slice_dims=(0,),
                                    start_index_map=(0,))
    return lax.gather(vec, idx, dn, (1,),
                      mode=lax.GatherScatterMode.PROMISE_IN_BOUNDS)


def _sc_aggregate_body(dual, n_chunks, x_hbm, esrc_hbm, edst_hbm,
                       ew_hbm, out_hbm, src_all, dst_all, w_all,
                       idx_a, idx_b, dst_a, dst_b, r16_a, r16_b,
                       out_a, out_b, acc, gsem_a, gsem_b, ssem_a, ssem_b,
                       stage_sem):
    c = lax.axis_index("c")
    s = lax.axis_index("s")

    zero = jnp.zeros((LANES,), jnp.float32)
    for r in range(CHUNK):
        for j in range(D // LANES):
            out_a[r, pl.ds(j * LANES, LANES)] = zero
    rows_pt = 624
    rem_rows = N - NS * rows_pt
    r0 = pl.multiple_of(s * rows_pt, 8)
    nfull = rows_pt // CHUNK
    tail = rows_pt - nfull * CHUNK
    for k in range(nfull):
        pltpu.sync_copy(out_a, acc.at[pl.ds(r0 + k * CHUNK, CHUNK)])
    if tail:
        pltpu.sync_copy(out_a.at[pl.ds(0, tail)],
                        acc.at[pl.ds(r0 + nfull * CHUNK, tail)])

    @pl.when(s == NS - 1)
    def _():
        pltpu.sync_copy(out_a.at[pl.ds(0, rem_rows)],
                        acc.at[pl.ds(NS * rows_pt, rem_rows)])

    plsc.subcore_barrier()

    ept = n_chunks * CHUNK
    if dual:
        tile_base = s * ept
        off = jnp.broadcast_to((c * N).astype(jnp.int32), (LANES,))
    else:
        tile_base = (c * NS + s) * ept

    n_segs = n_chunks // SEG_CHUNKS
    n_pairs = SEG_CHUNKS // 2

    def _prep_idx(ci, idx_v):
        base = pl.multiple_of(ci * CHUNK, 8)
        for g in range(CHUNK // LANES):
            sl = pl.ds(g * LANES, LANES)
            slb = pl.ds(base + g * LANES, LANES)
            if dual:
                idx_v[sl] = src_all[slb] + off
            else:
                idx_v[sl] = src_all[slb]

    def _prep_dst(ci, dst_v):
        base = pl.multiple_of(ci * CHUNK, 8)
        for g in range(CHUNK // LANES):
            sl = pl.ds(g * LANES, LANES)
            slb = pl.ds(base + g * LANES, LANES)
            dst_v[sl] = dst_all[slb]

    def _scale(ci, r16_v, out_v):
        base = pl.multiple_of(ci * CHUNK, 8)
        for g in range(CHUNK // LANES):
            w16 = w_all[pl.ds(base + g * LANES, LANES)]
            for i in range(LANES):
                w = _splat_lane(w16, i)
                r = g * LANES + i
                for j in range(D // LANES):
                    sl = pl.ds(j * LANES, LANES)
                    out_v[r, sl] = r16_v[r, sl] * w

    def _drain(out_v, dst_v, ssem):
        pltpu.make_async_copy(out_v, acc.at[dst_v], ssem).wait()

    def _stage(si, sem_or_none):
        sb = pl.multiple_of(tile_base + si * SEG_EDGES, 8)
        if sem_or_none is None:
            pltpu.sync_copy(esrc_hbm.at[pl.ds(sb, SEG_EDGES)], src_all)
            pltpu.sync_copy(edst_hbm.at[pl.ds(sb, SEG_EDGES)], dst_all)
            pltpu.sync_copy(ew_hbm.at[pl.ds(sb, SEG_EDGES)], w_all)
        else:
            pltpu.async_copy(esrc_hbm.at[pl.ds(sb, SEG_EDGES)], src_all,
                             sem_or_none)
            pltpu.async_copy(edst_hbm.at[pl.ds(sb, SEG_EDGES)], dst_all,
                             sem_or_none)
            pltpu.async_copy(ew_hbm.at[pl.ds(sb, SEG_EDGES)], w_all,
                             sem_or_none)

    def _stage_wait(si, sem):
        sb = pl.multiple_of(tile_base + si * SEG_EDGES, 8)
        pltpu.make_async_copy(esrc_hbm.at[pl.ds(sb, SEG_EDGES)], src_all,
                              sem).wait()
        pltpu.make_async_copy(edst_hbm.at[pl.ds(sb, SEG_EDGES)], dst_all,
                              sem).wait()
        pltpu.make_async_copy(ew_hbm.at[pl.ds(sb, SEG_EDGES)], w_all,
                              sem).wait()

    def seg_body(si, carry):
        _stage(si, None)
        _prep_idx(0, idx_a)
        pltpu.async_copy(x_hbm.at[idx_a], r16_a, gsem_a)
        _prep_idx(1, idx_b)
        pltpu.async_copy(x_hbm.at[idx_b], r16_b, gsem_b)

        def pair_body(pi, carry2):
            @pl.when(pi > 0)
            def _():
                _drain(out_a, dst_a, ssem_a)

            pltpu.make_async_copy(x_hbm.at[idx_a], r16_a, gsem_a).wait()
            _scale(2 * pi, r16_a, out_a)
            _prep_dst(2 * pi, dst_a)
            pltpu.async_copy(out_a, acc.at[dst_a], ssem_a, add=True)

            @pl.when(pi + 1 < n_pairs)
            def _():
                _prep_idx(2 * (pi + 1), idx_a)
                pltpu.async_copy(x_hbm.at[idx_a], r16_a, gsem_a)

            @pl.when(pi > 0)
            def _():
                _drain(out_b, dst_b, ssem_b)

            pltpu.make_async_copy(x_hbm.at[idx_b], r16_b, gsem_b).wait()
            _scale(2 * pi + 1, r16_b, out_b)
            _prep_dst(2 * pi + 1, dst_b)
            pltpu.async_copy(out_b, acc.at[dst_b], ssem_b, add=True)

            @pl.when(pi + 1 < n_pairs)
            def _():
                _prep_idx(2 * (pi + 1) + 1, idx_b)
                pltpu.async_copy(x_hbm.at[idx_b], r16_b, gsem_b)

            return carry2

        lax.fori_loop(0, n_pairs, pair_body, 0)
        _drain(out_a, dst_a, ssem_a)
        _drain(out_b, dst_b, ssem_b)
        rem_ci = SEG_CHUNKS - 1
        _prep_idx(rem_ci, idx_a)
        pltpu.async_copy(x_hbm.at[idx_a], r16_a, gsem_a).wait()
        _scale(rem_ci, r16_a, out_a)
        _prep_dst(rem_ci, dst_a)
        pltpu.sync_copy(out_a, acc.at[dst_a], add=True)
        return carry

    lax.fori_loop(0, n_segs, seg_body, 0)
    plsc.subcore_barrier()
    pltpu.sync_copy(acc.at[pl.ds(r0, rows_pt)],
                    out_hbm.at[c, pl.ds(r0, rows_pt)])

    @pl.when(s == NS - 1)
    def _():
        pltpu.sync_copy(acc.at[pl.ds(NS * rows_pt, rem_rows)],
                        out_hbm.at[c, pl.ds(NS * rows_pt, rem_rows)])


_SC_SCRATCH = [
    pltpu.VMEM((SEG_EDGES,), jnp.int32),
    pltpu.VMEM((SEG_EDGES,), jnp.int32),
    pltpu.VMEM((SEG_EDGES,), jnp.float32),
    pltpu.VMEM((CHUNK,), jnp.int32),
    pltpu.VMEM((CHUNK,), jnp.int32),
    pltpu.VMEM((CHUNK,), jnp.int32),
    pltpu.VMEM((CHUNK,), jnp.int32),
    pltpu.VMEM((CHUNK, D), jnp.float32),
    pltpu.VMEM((CHUNK, D), jnp.float32),
    pltpu.VMEM((CHUNK, D), jnp.float32),
    pltpu.VMEM((CHUNK, D), jnp.float32),
    pltpu.VMEM_SHARED((N, D), jnp.float32),
    pltpu.SemaphoreType.DMA,
    pltpu.SemaphoreType.DMA,
    pltpu.SemaphoreType.DMA,
    pltpu.SemaphoreType.DMA,
    pltpu.SemaphoreType.DMA,
]


def _sc_pass1(XHp, esrc, edst, ew):
    mesh = plsc.VectorSubcoreMesh(core_axis_name="c", subcore_axis_name="s")
    body = functools.partial(_sc_aggregate_body, True, E // NS // CHUNK)
    f = pl.kernel(body,
                  out_type=jax.ShapeDtypeStruct((NC, N, D), jnp.float32),
                  mesh=mesh, scratch_types=_SC_SCRATCH)
    return f(XHp, esrc, edst, ew)


def _sc_pass2(HRp, esrc, edst, ew):
    mesh = plsc.VectorSubcoreMesh(core_axis_name="c", subcore_axis_name="s")
    body = functools.partial(_sc_aggregate_body, False,
                             E // (NC * NS) // CHUNK)
    f = pl.kernel(body,
                  out_type=jax.ShapeDtypeStruct((NC, N, D), jnp.float32),
                  mesh=mesh, scratch_types=_SC_SCRATCH)
    return f(HRp, esrc, edst, ew)


def _tc_gates(aggrXH, X, H, Wzr, bzr):
    def body(axh, x, h, wzr, b, z_out, hr_out):
        a = jnp.concatenate([axh[0], x[...], axh[1], h[...]], axis=1)
        g = jnp.dot(a, wzr[...], preferred_element_type=jnp.float32) + b[...]
        z = jax.nn.sigmoid(g[:, :D])
        r = jax.nn.sigmoid(g[:, D:])
        z_out[...] = z
        hr_out[...] = h[...] * r

    return pl.pallas_call(
        body,
        grid=(N // BN,),
        in_specs=[
            pl.BlockSpec((NC, BN, D), lambda i: (0, i, 0)),
            pl.BlockSpec((BN, D), lambda i: (i, 0)),
            pl.BlockSpec((BN, D), lambda i: (i, 0)),
            pl.BlockSpec((4 * D, 2 * D), lambda i: (0, 0)),
            pl.BlockSpec((1, 2 * D), lambda i: (0, 0)),
        ],
        out_specs=[pl.BlockSpec((BN, D), lambda i: (i, 0))] * 2,
        out_shape=[jax.ShapeDtypeStruct((N, D), jnp.float32)] * 2,
    )(aggrXH, X, H, Wzr, bzr)


def _tc_out(aggrXH, parts, X, HR, H, Z, Wh, bh):
    def body(axh, p, x, hr, h, z, wh, b, out):
        a = jnp.concatenate([axh[0], x[...], p[0], p[1], hr[...]], axis=1)
        g = jnp.dot(a, wh[...], preferred_element_type=jnp.float32) + b[...]
        ht = jnp.tanh(g)
        out[...] = z[...] * h[...] + (1.0 - z[...]) * ht

    return pl.pallas_call(
        body,
        grid=(N // BN,),
        in_specs=[
            pl.BlockSpec((1, BN, D), lambda i: (0, i, 0)),
            pl.BlockSpec((NC, BN, D), lambda i: (0, i, 0)),
            pl.BlockSpec((BN, D), lambda i: (i, 0)),
            pl.BlockSpec((BN, D), lambda i: (i, 0)),
            pl.BlockSpec((BN, D), lambda i: (i, 0)),
            pl.BlockSpec((BN, D), lambda i: (i, 0)),
            pl.BlockSpec((5 * D, D), lambda i: (0, 0)),
            pl.BlockSpec((1, D), lambda i: (0, 0)),
        ],
        out_specs=pl.BlockSpec((BN, D), lambda i: (i, 0)),
        out_shape=jax.ShapeDtypeStruct((N, D), jnp.float32),
    )(aggrXH, parts, X, HR, H, Z, Wh, bh)


def kernel(X, edge_index, edge_weight, H,
           Wl_xz, Wr_xz, b_xz, Wl_hz, Wr_hz, b_hz,
           Wl_xr, Wr_xr, b_xr, Wl_hr, Wr_hr, b_hr,
           Wl_xh, Wr_xh, b_xh, Wl_hh, Wr_hh, b_hh):
    esrc = edge_index[0]
    edst = edge_index[1]
    XHp = jnp.concatenate([X, H], axis=0)
    aggrXH = _sc_pass1(XHp, esrc, edst, edge_weight)

    Wzr = jnp.concatenate([
        jnp.concatenate([Wl_xz, Wl_xr], axis=1),
        jnp.concatenate([Wr_xz, Wr_xr], axis=1),
        jnp.concatenate([Wl_hz, Wl_hr], axis=1),
        jnp.concatenate([Wr_hz, Wr_hr], axis=1),
    ], axis=0)
    bzr = jnp.concatenate([b_xz + b_hz, b_xr + b_hr]).reshape(1, 2 * D)
    Z, HR = _tc_gates(aggrXH, X, H, Wzr, bzr)

    parts = _sc_pass2(HR, esrc, edst, edge_weight)

    Wh = jnp.concatenate([Wl_xh, Wr_xh, Wl_hh, Wl_hh,
                          Wr_hh], axis=0)
    bh = (b_xh + b_hh).reshape(1, D)
    return _tc_out(aggrXH, parts, X, HR, H, Z, Wh, bh)


# speedup vs baseline: 1.0942x; 1.0036x over previous
import functools

import jax
import jax.numpy as jnp
from jax import lax
from jax.experimental import pallas as pl
from jax.experimental.pallas import tpu as pltpu
from jax.experimental.pallas import tpu_sc as plsc

N = 10000
E = 320000
D = 128
NC = 2
NS = 16
LANES = 16
CHUNK = 80
SEG_CHUNKS = 25
SEG_EDGES = SEG_CHUNKS * CHUNK
BN = 1000

def _splat_lane(vec, i):
    idx = jnp.full((LANES, 1), i, jnp.int32)
    dn = lax.GatherDimensionNumbers(offset_dims=(), collapsed_slice_dims=(0,),
                                    start_index_map=(0,))
    return lax.gather(vec, idx, dn, (1,),
                      mode=lax.GatherScatterMode.PROMISE_IN_BOUNDS)


def _sc_aggregate_body(dual, n_chunks, x_hbm, esrc_hbm, edst_hbm,
                       ew_hbm, out_hbm, src_all, dst_all, w_all,
                       idx_a, idx_b, dst_a, dst_b, gth_a, gth_b,
                       out_a, out_b, acc, gsem_a, gsem_b, ssem_a, ssem_b):
    c = lax.axis_index("c")
    s = lax.axis_index("s")

    zero = jnp.zeros((LANES,), jnp.float32)
    for r in range(CHUNK):
        for j in range(D // LANES):
            out_a[r, pl.ds(j * LANES, LANES)] = zero
    rows_pt = 624
    rem_rows = N - NS * rows_pt
    r0 = pl.multiple_of(s * rows_pt, 8)
    nfull = rows_pt // CHUNK
    tail = rows_pt - nfull * CHUNK
    for k in range(nfull):
        pltpu.sync_copy(out_a, acc.at[pl.ds(r0 + k * CHUNK, CHUNK)])
    if tail:
        pltpu.sync_copy(out_a.at[pl.ds(0, tail)],
                        acc.at[pl.ds(r0 + nfull * CHUNK, tail)])

    @pl.when(s == NS - 1)
    def _():
        pltpu.sync_copy(out_a.at[pl.ds(0, rem_rows)],
                        acc.at[pl.ds(NS * rows_pt, rem_rows)])

    plsc.subcore_barrier()

    ept = n_chunks * CHUNK
    if dual:
        tile_base = s * ept
        off = jnp.broadcast_to((c * N).astype(jnp.int32), (LANES,))
    else:
        tile_base = (c * NS + s) * ept

    n_segs = n_chunks // SEG_CHUNKS
    n_pairs = SEG_CHUNKS // 2

    def _prep_idx(ci, idx_v):
        base = pl.multiple_of(ci * CHUNK, 8)
        for g in range(CHUNK // LANES):
            sl = pl.ds(g * LANES, LANES)
            slb = pl.ds(base + g * LANES, LANES)
            if dual:
                idx_v[sl] = src_all[slb] + off
            else:
                idx_v[sl] = src_all[slb]

    def _prep_dst(ci, dst_v):
        base = pl.multiple_of(ci * CHUNK, 8)
        for g in range(CHUNK // LANES):
            sl = pl.ds(g * LANES, LANES)
            slb = pl.ds(base + g * LANES, LANES)
            dst_v[sl] = dst_all[slb]

    def _scale(ci, gth_v, out_v):
        base = pl.multiple_of(ci * CHUNK, 8)
        for g in range(CHUNK // LANES):
            w16 = w_all[pl.ds(base + g * LANES, LANES)]
            for i in range(LANES):
                w = _splat_lane(w16, i)
                r = g * LANES + i
                for j in range(D // LANES):
                    sl = pl.ds(j * LANES, LANES)
                    out_v[r, sl] = gth_v[r, sl] * w

    def _drain(out_v, dst_v, ssem):
        pltpu.make_async_copy(out_v, acc.at[dst_v], ssem).wait()

    def _stage(si):
        sb = pl.multiple_of(tile_base + si * SEG_EDGES, 8)
        pltpu.sync_copy(esrc_hbm.at[pl.ds(sb, SEG_EDGES)], src_all)
        pltpu.sync_copy(edst_hbm.at[pl.ds(sb, SEG_EDGES)], dst_all)
        pltpu.sync_copy(ew_hbm.at[pl.ds(sb, SEG_EDGES)], w_all)

    def seg_body(si, carry):
        _stage(si)
        _prep_idx(0, idx_a)
        pltpu.async_copy(x_hbm.at[idx_a], gth_a, gsem_a)
        _prep_idx(1, idx_b)
        pltpu.async_copy(x_hbm.at[idx_b], gth_b, gsem_b)

        def pair_body(pi, carry2):
            @pl.when(pi > 0)
            def _():
                _drain(out_a, dst_a, ssem_a)

            pltpu.make_async_copy(x_hbm.at[idx_a], gth_a, gsem_a).wait()
            _scale(2 * pi, gth_a, out_a)
            _prep_dst(2 * pi, dst_a)
            pltpu.async_copy(out_a, acc.at[dst_a], ssem_a, add=True)

            @pl.when(pi + 1 < n_pairs)
            def _():
                _prep_idx(2 * (pi + 1), idx_a)
                pltpu.async_copy(x_hbm.at[idx_a], gth_a, gsem_a)

            @pl.when(pi > 0)
            def _():
                _drain(out_b, dst_b, ssem_b)

            pltpu.make_async_copy(x_hbm.at[idx_b], gth_b, gsem_b).wait()
            _scale(2 * pi + 1, gth_b, out_b)
            _prep_dst(2 * pi + 1, dst_b)
            pltpu.async_copy(out_b, acc.at[dst_b], ssem_b, add=True)

            @pl.when(pi + 1 < n_pairs)
            def _():
                _prep_idx(2 * (pi + 1) + 1, idx_b)
                pltpu.async_copy(x_hbm.at[idx_b], gth_b, gsem_b)

            return carry2

        lax.fori_loop(0, n_pairs, pair_body, 0)
        _drain(out_a, dst_a, ssem_a)
        _drain(out_b, dst_b, ssem_b)
        rem_ci = SEG_CHUNKS - 1
        _prep_idx(rem_ci, idx_a)
        pltpu.async_copy(x_hbm.at[idx_a], gth_a, gsem_a).wait()
        _scale(rem_ci, gth_a, out_a)
        _prep_dst(rem_ci, dst_a)
        pltpu.sync_copy(out_a, acc.at[dst_a], add=True)
        return carry

    lax.fori_loop(0, n_segs, seg_body, 0)
    plsc.subcore_barrier()
    pltpu.sync_copy(acc.at[pl.ds(r0, rows_pt)],
                    out_hbm.at[c, pl.ds(r0, rows_pt)])

    @pl.when(s == NS - 1)
    def _():
        pltpu.sync_copy(acc.at[pl.ds(NS * rows_pt, rem_rows)],
                        out_hbm.at[c, pl.ds(NS * rows_pt, rem_rows)])


_SC_SCRATCH = [
    pltpu.VMEM((SEG_EDGES,), jnp.int32),
    pltpu.VMEM((SEG_EDGES,), jnp.int32),
    pltpu.VMEM((SEG_EDGES,), jnp.float32),
    pltpu.VMEM((CHUNK,), jnp.int32),
    pltpu.VMEM((CHUNK,), jnp.int32),
    pltpu.VMEM((CHUNK,), jnp.int32),
    pltpu.VMEM((CHUNK,), jnp.int32),
    pltpu.VMEM((CHUNK, D), jnp.float32),
    pltpu.VMEM((CHUNK, D), jnp.float32),
    pltpu.VMEM((CHUNK, D), jnp.float32),
    pltpu.VMEM((CHUNK, D), jnp.float32),
    pltpu.VMEM_SHARED((N, D), jnp.float32),
    pltpu.SemaphoreType.DMA,
    pltpu.SemaphoreType.DMA,
    pltpu.SemaphoreType.DMA,
    pltpu.SemaphoreType.DMA,
]


def _sc_pass1(XHp, esrc, edst, ew):
    mesh = plsc.VectorSubcoreMesh(core_axis_name="c", subcore_axis_name="s")
    body = functools.partial(_sc_aggregate_body, True, E // NS // CHUNK)
    f = pl.kernel(body,
                  out_type=jax.ShapeDtypeStruct((NC, N, D), jnp.float32),
                  mesh=mesh, scratch_types=_SC_SCRATCH)
    return f(XHp, esrc, edst, ew)


def _sc_pass2(HRp, esrc, edst, ew):
    mesh = plsc.VectorSubcoreMesh(core_axis_name="c", subcore_axis_name="s")
    body = functools.partial(_sc_aggregate_body, False,
                             E // (NC * NS) // CHUNK)
    f = pl.kernel(body,
                  out_type=jax.ShapeDtypeStruct((NC, N, D), jnp.float32),
                  mesh=mesh, scratch_types=_SC_SCRATCH)
    return f(HRp, esrc, edst, ew)


def _tc_gates(aggrXH, X, H, Wzr, bzr):
    def body(axh, x, h, wzr, b, z_out, hr_out):
        a = jnp.concatenate([axh[0], x[...], axh[1], h[...]], axis=1)
        g = jnp.dot(a, wzr[...], preferred_element_type=jnp.float32) + b[...]
        z = jax.nn.sigmoid(g[:, :D])
        r = jax.nn.sigmoid(g[:, D:])
        z_out[...] = z
        hr_out[...] = h[...] * r

    return pl.pallas_call(
        body,
        grid=(N // BN,),
        in_specs=[
            pl.BlockSpec((NC, BN, D), lambda i: (0, i, 0)),
            pl.BlockSpec((BN, D), lambda i: (i, 0)),
            pl.BlockSpec((BN, D), lambda i: (i, 0)),
            pl.BlockSpec((4 * D, 2 * D), lambda i: (0, 0)),
            pl.BlockSpec((1, 2 * D), lambda i: (0, 0)),
        ],
        out_specs=[pl.BlockSpec((BN, D), lambda i: (i, 0))] * 2,
        out_shape=[jax.ShapeDtypeStruct((N, D), jnp.float32)] * 2,
    )(aggrXH, X, H, Wzr, bzr)


def _tc_out(aggrXH, parts, X, HR, H, Z, Wh, bh):
    def body(axh, p, x, hr, h, z, wh, b, out):
        a = jnp.concatenate([axh[0], x[...], p[0], p[1], hr[...]], axis=1)
        g = jnp.dot(a, wh[...], preferred_element_type=jnp.float32) + b[...]
        ht = jnp.tanh(g)
        out[...] = z[...] * h[...] + (1.0 - z[...]) * ht

    return pl.pallas_call(
        body,
        grid=(N // BN,),
        in_specs=[
            pl.BlockSpec((1, BN, D), lambda i: (0, i, 0)),
            pl.BlockSpec((NC, BN, D), lambda i: (0, i, 0)),
            pl.BlockSpec((BN, D), lambda i: (i, 0)),
            pl.BlockSpec((BN, D), lambda i: (i, 0)),
            pl.BlockSpec((BN, D), lambda i: (i, 0)),
            pl.BlockSpec((BN, D), lambda i: (i, 0)),
            pl.BlockSpec((5 * D, D), lambda i: (0, 0)),
            pl.BlockSpec((1, D), lambda i: (0, 0)),
        ],
        out_specs=pl.BlockSpec((BN, D), lambda i: (i, 0)),
        out_shape=jax.ShapeDtypeStruct((N, D), jnp.float32),
    )(aggrXH, parts, X, HR, H, Z, Wh, bh)


def kernel(X, edge_index, edge_weight, H,
           Wl_xz, Wr_xz, b_xz, Wl_hz, Wr_hz, b_hz,
           Wl_xr, Wr_xr, b_xr, Wl_hr, Wr_hr, b_hr,
           Wl_xh, Wr_xh, b_xh, Wl_hh, Wr_hh, b_hh):
    esrc = edge_index[0]
    edst = edge_index[1]
    XHp = jnp.concatenate([X, H], axis=0)
    aggrXH = _sc_pass1(XHp, esrc, edst, edge_weight)

    Wzr = jnp.concatenate([
        jnp.concatenate([Wl_xz, Wl_xr], axis=1),
        jnp.concatenate([Wr_xz, Wr_xr], axis=1),
        jnp.concatenate([Wl_hz, Wl_hr], axis=1),
        jnp.concatenate([Wr_hz, Wr_hr], axis=1),
    ], axis=0)
    bzr = jnp.concatenate([b_xz + b_hz, b_xr + b_hr]).reshape(1, 2 * D)
    Z, HR = _tc_gates(aggrXH, X, H, Wzr, bzr)

    parts = _sc_pass2(HR, esrc, edst, edge_weight)

    Wh = jnp.concatenate([Wl_xh, Wr_xh, Wl_hh, Wl_hh,
                          Wr_hh], axis=0)
    bh = (b_xh + b_hh).reshape(1, D)
    return _tc_out(aggrXH, parts, X, HR, H, Z, Wh, bh)


# speedup vs baseline: 1.1012x; 1.0064x over previous
import functools

import jax
import jax.numpy as jnp
from jax import lax
from jax.experimental import pallas as pl
from jax.experimental.pallas import tpu as pltpu
from jax.experimental.pallas import tpu_sc as plsc

N = 10000
E = 320000
D = 128
NC = 2
NS = 16
LANES = 16
CHUNK = 80
SEG_CHUNKS = 25
SEG_EDGES = SEG_CHUNKS * CHUNK
BN = 1000

def _splat_lane(vec, i):
    idx = jnp.full((LANES, 1), i, jnp.int32)
    dn = lax.GatherDimensionNumbers(offset_dims=(), collapsed_slice_dims=(0,),
                                    start_index_map=(0,))
    return lax.gather(vec, idx, dn, (1,),
                      mode=lax.GatherScatterMode.PROMISE_IN_BOUNDS)


def _sc_aggregate_body(dual, n_chunks, x_hbm, esrc_hbm, edst_hbm,
                       ew_hbm, out_hbm, src_all, dst_all, w_all,
                       idx_a, idx_b, dst_a, dst_b, gth_a, gth_b,
                       out_a, out_b, acc, gsem_a, gsem_b, ssem_a, ssem_b):
    c = lax.axis_index("c")
    s = lax.axis_index("s")

    zero = jnp.zeros((LANES,), jnp.float32)
    for r in range(CHUNK):
        for j in range(D // LANES):
            out_a[r, pl.ds(j * LANES, LANES)] = zero
    rows_pt = 624
    rem_rows = N - NS * rows_pt
    r0 = pl.multiple_of(s * rows_pt, 8)
    nfull = rows_pt // CHUNK
    tail = rows_pt - nfull * CHUNK
    for k in range(nfull):
        pltpu.sync_copy(out_a, acc.at[pl.ds(r0 + k * CHUNK, CHUNK)])
    if tail:
        pltpu.sync_copy(out_a.at[pl.ds(0, tail)],
                        acc.at[pl.ds(r0 + nfull * CHUNK, tail)])

    @pl.when(s == NS - 1)
    def _():
        pltpu.sync_copy(out_a.at[pl.ds(0, rem_rows)],
                        acc.at[pl.ds(NS * rows_pt, rem_rows)])

    plsc.subcore_barrier()

    ept = n_chunks * CHUNK
    if dual:
        tile_base = s * ept
        off = jnp.broadcast_to((c * N).astype(jnp.int32), (LANES,))
    else:
        tile_base = (c * NS + s) * ept

    n_segs = n_chunks // SEG_CHUNKS
    n_pairs = SEG_CHUNKS // 2

    def _prep_idx(ci, idx_v):
        base = pl.multiple_of(ci * CHUNK, 8)
        for g in range(CHUNK // LANES):
            sl = pl.ds(g * LANES, LANES)
            slb = pl.ds(base + g * LANES, LANES)
            if dual:
                idx_v[sl] = src_all[slb] + off
            else:
                idx_v[sl] = src_all[slb]

    def _prep_dst(ci, dst_v):
        base = pl.multiple_of(ci * CHUNK, 8)
        for g in range(CHUNK // LANES):
            sl = pl.ds(g * LANES, LANES)
            slb = pl.ds(base + g * LANES, LANES)
            dst_v[sl] = dst_all[slb]

    def _scale(ci, gth_v, out_v):
        base = pl.multiple_of(ci * CHUNK, 8)
        for g in range(CHUNK // LANES):
            w16 = w_all[pl.ds(base + g * LANES, LANES)]
            for i in range(LANES):
                w = _splat_lane(w16, i)
                r = g * LANES + i
                for j in range(D // LANES):
                    sl = pl.ds(j * LANES, LANES)
                    out_v[r, sl] = gth_v[r, sl] * w

    def _drain(out_v, dst_v, ssem):
        pltpu.make_async_copy(out_v, acc.at[dst_v], ssem).wait()

    def _stage(si):
        sb = pl.multiple_of(tile_base + si * SEG_EDGES, 8)
        pltpu.sync_copy(esrc_hbm.at[pl.ds(sb, SEG_EDGES)], src_all)
        pltpu.sync_copy(edst_hbm.at[pl.ds(sb, SEG_EDGES)], dst_all)
        pltpu.sync_copy(ew_hbm.at[pl.ds(sb, SEG_EDGES)], w_all)

    def seg_body(si, carry):
        _stage(si)
        _prep_idx(0, idx_a)
        pltpu.async_copy(x_hbm.at[idx_a], gth_a, gsem_a)
        _prep_idx(1, idx_b)
        pltpu.async_copy(x_hbm.at[idx_b], gth_b, gsem_b)

        def pair_body(pi, carry2):
            @pl.when(pi > 0)
            def _():
                _drain(out_a, dst_a, ssem_a)

            _prep_dst(2 * pi, dst_a)
            pltpu.make_async_copy(x_hbm.at[idx_a], gth_a, gsem_a).wait()
            _scale(2 * pi, gth_a, out_a)
            pltpu.async_copy(out_a, acc.at[dst_a], ssem_a, add=True)

            @pl.when(pi + 1 < n_pairs)
            def _():
                _prep_idx(2 * (pi + 1), idx_a)
                pltpu.async_copy(x_hbm.at[idx_a], gth_a, gsem_a)

            @pl.when(pi > 0)
            def _():
                _drain(out_b, dst_b, ssem_b)

            _prep_dst(2 * pi + 1, dst_b)
            pltpu.make_async_copy(x_hbm.at[idx_b], gth_b, gsem_b).wait()
            _scale(2 * pi + 1, gth_b, out_b)
            pltpu.async_copy(out_b, acc.at[dst_b], ssem_b, add=True)

            @pl.when(pi + 1 < n_pairs)
            def _():
                _prep_idx(2 * (pi + 1) + 1, idx_b)
                pltpu.async_copy(x_hbm.at[idx_b], gth_b, gsem_b)

            return carry2

        lax.fori_loop(0, n_pairs, pair_body, 0)
        _drain(out_a, dst_a, ssem_a)
        _drain(out_b, dst_b, ssem_b)
        rem_ci = SEG_CHUNKS - 1
        _prep_idx(rem_ci, idx_a)
        pltpu.async_copy(x_hbm.at[idx_a], gth_a, gsem_a).wait()
        _scale(rem_ci, gth_a, out_a)
        _prep_dst(rem_ci, dst_a)
        pltpu.sync_copy(out_a, acc.at[dst_a], add=True)
        return carry

    lax.fori_loop(0, n_segs, seg_body, 0)
    plsc.subcore_barrier()
    pltpu.sync_copy(acc.at[pl.ds(r0, rows_pt)],
                    out_hbm.at[c, pl.ds(r0, rows_pt)])

    @pl.when(s == NS - 1)
    def _():
        pltpu.sync_copy(acc.at[pl.ds(NS * rows_pt, rem_rows)],
                        out_hbm.at[c, pl.ds(NS * rows_pt, rem_rows)])


_SC_SCRATCH = [
    pltpu.VMEM((SEG_EDGES,), jnp.int32),
    pltpu.VMEM((SEG_EDGES,), jnp.int32),
    pltpu.VMEM((SEG_EDGES,), jnp.float32),
    pltpu.VMEM((CHUNK,), jnp.int32),
    pltpu.VMEM((CHUNK,), jnp.int32),
    pltpu.VMEM((CHUNK,), jnp.int32),
    pltpu.VMEM((CHUNK,), jnp.int32),
    pltpu.VMEM((CHUNK, D), jnp.float32),
    pltpu.VMEM((CHUNK, D), jnp.float32),
    pltpu.VMEM((CHUNK, D), jnp.float32),
    pltpu.VMEM((CHUNK, D), jnp.float32),
    pltpu.VMEM_SHARED((N, D), jnp.float32),
    pltpu.SemaphoreType.DMA,
    pltpu.SemaphoreType.DMA,
    pltpu.SemaphoreType.DMA,
    pltpu.SemaphoreType.DMA,
]


def _sc_pass1(XHp, esrc, edst, ew):
    mesh = plsc.VectorSubcoreMesh(core_axis_name="c", subcore_axis_name="s")
    body = functools.partial(_sc_aggregate_body, True, E // NS // CHUNK)
    f = pl.kernel(body,
                  out_type=jax.ShapeDtypeStruct((NC, N, D), jnp.float32),
                  mesh=mesh, scratch_types=_SC_SCRATCH)
    return f(XHp, esrc, edst, ew)


def _sc_pass2(HRp, esrc, edst, ew):
    mesh = plsc.VectorSubcoreMesh(core_axis_name="c", subcore_axis_name="s")
    body = functools.partial(_sc_aggregate_body, False,
                             E // (NC * NS) // CHUNK)
    f = pl.kernel(body,
                  out_type=jax.ShapeDtypeStruct((NC, N, D), jnp.float32),
                  mesh=mesh, scratch_types=_SC_SCRATCH)
    return f(HRp, esrc, edst, ew)


def _tc_gates(aggrXH, X, H, Wzr, bzr):
    def body(axh, x, h, wzr, b, z_out, hr_out):
        a = jnp.concatenate([axh[0], x[...], axh[1], h[...]], axis=1)
        g = jnp.dot(a, wzr[...], preferred_element_type=jnp.float32) + b[...]
        z = jax.nn.sigmoid(g[:, :D])
        r = jax.nn.sigmoid(g[:, D:])
        z_out[...] = z
        hr_out[...] = h[...] * r

    return pl.pallas_call(
        body,
        grid=(N // BN,),
        in_specs=[
            pl.BlockSpec((NC, BN, D), lambda i: (0, i, 0)),
            pl.BlockSpec((BN, D), lambda i: (i, 0)),
            pl.BlockSpec((BN, D), lambda i: (i, 0)),
            pl.BlockSpec((4 * D, 2 * D), lambda i: (0, 0)),
            pl.BlockSpec((1, 2 * D), lambda i: (0, 0)),
        ],
        out_specs=[pl.BlockSpec((BN, D), lambda i: (i, 0))] * 2,
        out_shape=[jax.ShapeDtypeStruct((N, D), jnp.float32)] * 2,
    )(aggrXH, X, H, Wzr, bzr)


def _tc_out(aggrXH, parts, X, HR, H, Z, Wh, bh):
    def body(axh, p, x, hr, h, z, wh, b, out):
        a = jnp.concatenate([axh[0], x[...], p[0], p[1], hr[...]], axis=1)
        g = jnp.dot(a, wh[...], preferred_element_type=jnp.float32) + b[...]
        ht = jnp.tanh(g)
        out[...] = z[...] * h[...] + (1.0 - z[...]) * ht

    return pl.pallas_call(
        body,
        grid=(N // BN,),
        in_specs=[
            pl.BlockSpec((1, BN, D), lambda i: (0, i, 0)),
            pl.BlockSpec((NC, BN, D), lambda i: (0, i, 0)),
            pl.BlockSpec((BN, D), lambda i: (i, 0)),
            pl.BlockSpec((BN, D), lambda i: (i, 0)),
            pl.BlockSpec((BN, D), lambda i: (i, 0)),
            pl.BlockSpec((BN, D), lambda i: (i, 0)),
            pl.BlockSpec((5 * D, D), lambda i: (0, 0)),
            pl.BlockSpec((1, D), lambda i: (0, 0)),
        ],
        out_specs=pl.BlockSpec((BN, D), lambda i: (i, 0)),
        out_shape=jax.ShapeDtypeStruct((N, D), jnp.float32),
    )(aggrXH, parts, X, HR, H, Z, Wh, bh)


def kernel(X, edge_index, edge_weight, H,
           Wl_xz, Wr_xz, b_xz, Wl_hz, Wr_hz, b_hz,
           Wl_xr, Wr_xr, b_xr, Wl_hr, Wr_hr, b_hr,
           Wl_xh, Wr_xh, b_xh, Wl_hh, Wr_hh, b_hh):
    esrc = edge_index[0]
    edst = edge_index[1]
    XHp = jnp.concatenate([X, H], axis=0)
    aggrXH = _sc_pass1(XHp, esrc, edst, edge_weight)

    Wzr = jnp.concatenate([
        jnp.concatenate([Wl_xz, Wl_xr], axis=1),
        jnp.concatenate([Wr_xz, Wr_xr], axis=1),
        jnp.concatenate([Wl_hz, Wl_hr], axis=1),
        jnp.concatenate([Wr_hz, Wr_hr], axis=1),
    ], axis=0)
    bzr = jnp.concatenate([b_xz + b_hz, b_xr + b_hr]).reshape(1, 2 * D)
    Z, HR = _tc_gates(aggrXH, X, H, Wzr, bzr)

    parts = _sc_pass2(HR, esrc, edst, edge_weight)

    Wh = jnp.concatenate([Wl_xh, Wr_xh, Wl_hh, Wl_hh,
                          Wr_hh], axis=0)
    bh = (b_xh + b_hh).reshape(1, D)
    return _tc_out(aggrXH, parts, X, HR, H, Z, Wh, bh)
